# baseline (speedup 1.0000x reference)
import functools

import jax
import jax.numpy as jnp
from jax import lax
from jax.experimental import pallas as pl
from jax.experimental.pallas import tpu as pltpu
from jax.experimental.pallas import tpu_sc as plsc

_D = 768
_LANES = 16
_NC = 2
_NS = 16
_NW = _NC * _NS
_C = 16
_NBUF = 4


def _emb_body(tok_hbm, posflat_hbm, tidx_hbm, pidx_hbm, out_hbm,
              posvm, tidx_v, pidx_v, t0, t1, t2, t3,
              g0, g1, g2, g3, s0, s1, s2, s3, *, per_w):
    wid = lax.axis_index("s") * _NC + lax.axis_index("c")
    base = wid * per_w
    nch = per_w // _C
    bufs = (t0, t1, t2, t3)
    gsem = (g0, g1, g2, g3)
    ssem = (s0, s1, s2, s3)
    rowi = lax.iota(jnp.int32, _LANES)
    zeros = jnp.zeros((_LANES,), jnp.int32)

    pltpu.sync_copy(posflat_hbm, posvm)
    pltpu.sync_copy(tidx_hbm.at[pl.ds(base, per_w)], tidx_v)
    pltpu.sync_copy(pidx_hbm.at[pl.ds(base, per_w)], pidx_v)

    def issue_gather(ci, s):
        pltpu.async_copy(tok_hbm.at[tidx_v.at[pl.ds(ci * _C, _C)]],
                         bufs[s], gsem[s])

    def wait_gather(ci, s):
        pltpu.make_async_copy(tok_hbm.at[tidx_v.at[pl.ds(ci * _C, _C)]],
                              bufs[s], gsem[s]).wait()

    def issue_scatter(ci, s):
        pltpu.async_copy(bufs[s], out_hbm.at[pl.ds(base + ci * _C, _C)],
                         ssem[s])

    def wait_scatter(ci, s):
        pltpu.make_async_copy(bufs[s],
                              out_hbm.at[pl.ds(base + ci * _C, _C)],
                              ssem[s]).wait()

    def add_chunk(ci, s):
        pvec = pidx_v[pl.ds(ci * _C, _LANES)] * _D

        def cbody(c, _):
            pv = plsc.load_gather(posvm, [pvec + c])
            plsc.addupdate_scatter(bufs[s], [rowi, zeros + c], pv)
            return ()

        lax.fori_loop(0, _D, cbody, (), unroll=8)

    issue_gather(0, 0)
    issue_gather(1, 1)
    for ci in (0, 1):
        issue_gather(ci + 2, ci + 2)
        wait_gather(ci, ci)
        add_chunk(ci, ci)
        issue_scatter(ci, ci)

    def outer(g, _):
        for sp in range(_NBUF):
            ci = 2 + g * _NBUF + sp
            s = (2 + sp) % _NBUF
            s2 = sp
            wait_scatter(ci - 2, s2)
            issue_gather(ci + 2, s2)
            wait_gather(ci, s)
            add_chunk(ci, s)
            issue_scatter(ci, s)
        return ()

    lax.fori_loop(0, (nch - 4) // _NBUF, outer, (), unroll=False)

    for ci in (nch - 2, nch - 1):
        s = ci % _NBUF
        wait_scatter(ci - 2, (ci + 2) % _NBUF)
        wait_gather(ci, s)
        add_chunk(ci, s)
        issue_scatter(ci, s)
    for ci in (nch - 2, nch - 1):
        wait_scatter(ci, ci % _NBUF)


def kernel(token_table, pos_table, tokens, positions):
    b, l = tokens.shape
    bt = b * l
    per_w = bt // _NW
    assert per_w % _C == 0 and (per_w // _C) % _NBUF == 0

    tidx = tokens.reshape(bt).astype(jnp.int32)
    pidx = positions.reshape(bt).astype(jnp.int32)
    posflat = pos_table.reshape(-1)

    mesh = plsc.VectorSubcoreMesh(core_axis_name="c", subcore_axis_name="s")
    body = functools.partial(_emb_body, per_w=per_w)
    run = pl.kernel(
        body,
        mesh=mesh,
        compiler_params=pltpu.CompilerParams(needs_layout_passes=False),
        out_type=jax.ShapeDtypeStruct((bt, _D), jnp.float32),
        scratch_types=[
            pltpu.VMEM((pos_table.size,), jnp.float32),
            pltpu.VMEM((per_w,), jnp.int32),
            pltpu.VMEM((per_w,), jnp.int32),
        ] + [pltpu.VMEM((_C, _D), jnp.float32)] * _NBUF
          + [pltpu.SemaphoreType.DMA] * (2 * _NBUF),
    )
    out = run(token_table, posflat, tidx, pidx)
    return out.reshape(b, l, _D)


# speedup vs baseline: 3.4079x; 3.4079x over previous
import functools

import jax
import jax.numpy as jnp
from jax import lax
from jax.experimental import pallas as pl
from jax.experimental.pallas import tpu as pltpu
from jax.experimental.pallas import tpu_sc as plsc

_D = 768
_LANES = 16
_NC = 2
_NS = 16
_NW = _NC * _NS
_C = 16
_NBUF = 4


def _emb_body(tok_hbm, pos_hbm, tidx_hbm, pidx_hbm, out_hbm,
              tidx_v, pidx_v,
              t0, t1, t2, t3, p0, p1, p2, p3,
              gt0, gt1, gt2, gt3, gp0, gp1, gp2, gp3,
              s0, s1, s2, s3, *, per_w):
    wid = lax.axis_index("s") * _NC + lax.axis_index("c")
    base = wid * per_w
    nch = per_w // _C
    tbuf = (t0, t1, t2, t3)
    pbuf = (p0, p1, p2, p3)
    gtsem = (gt0, gt1, gt2, gt3)
    gpsem = (gp0, gp1, gp2, gp3)
    ssem = (s0, s1, s2, s3)

    pltpu.sync_copy(tidx_hbm.at[pl.ds(base, per_w)], tidx_v)
    pltpu.sync_copy(pidx_hbm.at[pl.ds(base, per_w)], pidx_v)

    def issue_gathers(ci, s):
        pltpu.async_copy(tok_hbm.at[tidx_v.at[pl.ds(ci * _C, _C)]],
                         tbuf[s], gtsem[s])
        pltpu.async_copy(pos_hbm.at[pidx_v.at[pl.ds(ci * _C, _C)]],
                         pbuf[s], gpsem[s])

    def wait_gathers(ci, s):
        pltpu.make_async_copy(tok_hbm.at[tidx_v.at[pl.ds(ci * _C, _C)]],
                              tbuf[s], gtsem[s]).wait()
        pltpu.make_async_copy(pos_hbm.at[pidx_v.at[pl.ds(ci * _C, _C)]],
                              pbuf[s], gpsem[s]).wait()

    def issue_scatter(ci, s):
        pltpu.async_copy(tbuf[s], out_hbm.at[pl.ds(base + ci * _C, _C)],
                         ssem[s])

    def wait_scatter(ci, s):
        pltpu.make_async_copy(tbuf[s],
                              out_hbm.at[pl.ds(base + ci * _C, _C)],
                              ssem[s]).wait()

    def add_chunk(s):
        t = tbuf[s]
        p = pbuf[s]

        @plsc.parallel_loop(0, _C, step=1, unroll=2)
        def _(i):
            for j in range(_D // _LANES):
                sl = pl.ds(j * _LANES, _LANES)
                plsc.addupdate(t.at[i, sl], p[i, sl])

    issue_gathers(0, 0)
    issue_gathers(1, 1)
    for ci in (0, 1):
        issue_gathers(ci + 2, ci + 2)
        wait_gathers(ci, ci)
        add_chunk(ci)
        issue_scatter(ci, ci)

    def outer(g, _):
        for sp in range(_NBUF):
            ci = 2 + g * _NBUF + sp
            s = (2 + sp) % _NBUF
            s2 = sp
            wait_scatter(ci - 2, s2)
            issue_gathers(ci + 2, s2)
            wait_gathers(ci, s)
            add_chunk(s)
            issue_scatter(ci, s)
        return ()

    lax.fori_loop(0, (nch - 4) // _NBUF, outer, (), unroll=False)

    for ci in (nch - 2, nch - 1):
        s = ci % _NBUF
        wait_scatter(ci - 2, (ci + 2) % _NBUF)
        wait_gathers(ci, s)
        add_chunk(s)
        issue_scatter(ci, s)
    for ci in (nch - 2, nch - 1):
        wait_scatter(ci, ci % _NBUF)


def kernel(token_table, pos_table, tokens, positions):
    b, l = tokens.shape
    bt = b * l
    per_w = bt // _NW
    assert per_w % _C == 0 and (per_w // _C) % _NBUF == 0

    tidx = tokens.reshape(bt).astype(jnp.int32)
    pidx = positions.reshape(bt).astype(jnp.int32)

    mesh = plsc.VectorSubcoreMesh(core_axis_name="c", subcore_axis_name="s")
    body = functools.partial(_emb_body, per_w=per_w)
    run = pl.kernel(
        body,
        mesh=mesh,
        compiler_params=pltpu.CompilerParams(needs_layout_passes=False),
        out_type=jax.ShapeDtypeStruct((bt, _D), jnp.float32),
        scratch_types=[
            pltpu.VMEM((per_w,), jnp.int32),
            pltpu.VMEM((per_w,), jnp.int32),
        ] + [pltpu.VMEM((_C, _D), jnp.float32)] * (2 * _NBUF)
          + [pltpu.SemaphoreType.DMA] * (3 * _NBUF),
    )
    out = run(token_table, pos_table, tidx, pidx)
    return out.reshape(b, l, _D)
